# Initial kernel scaffold; baseline (speedup 1.0000x reference)
import jax
import jax.numpy as jnp
from jax.experimental import pallas as pl


def kernel(updates, mask, output):
    raise NotImplementedError("write your pallas kernel here")


# speedup vs baseline: 9.7299x; 9.7299x over previous
import jax
import jax.numpy as jnp
from jax import lax
from jax.experimental import pallas as pl
from jax.experimental.pallas import tpu as pltpu
from jax.experimental.pallas import tpu_sc as plsc

B = 4
H = 192
W = 192
C = 96
HOUT = 384
WOUT = 384
HWIN = H * W
POUT = HOUT * WOUT
CB = 6
NCB = C // CB
NUNITS = B * NCB
NSC = 2
NTILE = 16
UNITS_PER_SC = NUNITS // NSC
NELEM_T = HWIN * CB // NTILE
ACC_N = POUT * CB
ACC_T = ACC_N // NTILE
LANES = 16


def _sc_body(upd_hbm, mask_hbm, out_hbm, upd_v, mask_v, idx_v, zbuf, acc):
    cid = lax.axis_index("c")
    sid = lax.axis_index("s")

    def _zero(i, carry):
        zbuf[pl.ds(i * LANES, LANES)] = jnp.zeros((LANES,), jnp.float32)
        return carry

    lax.fori_loop(0, NELEM_T // LANES, _zero, 0)

    a0 = sid * ACC_T
    e0 = sid * NELEM_T

    def _unit(u, carry):
        unit = cid * UNITS_PER_SC + u
        b = unit // NCB
        cb = unit % NCB

        for k in range(ACC_T // NELEM_T):
            pltpu.sync_copy(zbuf, acc.at[pl.ds(a0 + k * NELEM_T, NELEM_T)])
        plsc.subcore_barrier()

        pltpu.sync_copy(mask_hbm.at[b, cb, pl.ds(e0, NELEM_T)], mask_v)
        pltpu.sync_copy(upd_hbm.at[b, cb, pl.ds(e0, NELEM_T)], upd_v)

        lanes = lax.iota(jnp.int32, LANES)
        cls = []
        for k in range(3):
            t = lanes + (k * LANES) % CB
            for s in range((LANES + CB - 1) // CB, 0, -1):
                t = jnp.where(t >= s * CB, t - s * CB, t)
            cls.append(t)

        def _mkidx(i, carry2):
            for k in range(3):
                base = i * (3 * LANES) + k * LANES
                m = mask_v[pl.ds(base, LANES)]
                n = lax.shift_right_logical(m, 5)
                q = (
                    n.astype(jnp.float32) * jnp.float32(1.0 / 3.0)
                    + jnp.float32(0.5)
                ).astype(jnp.int32)
                rr = n - q * 3
                q = q + lax.shift_right_arithmetic(rr, 31)
                idx_v[pl.ds(base, LANES)] = q * CB + cls[k]
            return carry2

        lax.fori_loop(0, NELEM_T // (3 * LANES), _mkidx, 0)

        pltpu.sync_copy(upd_v, acc.at[idx_v], add=True)
        plsc.subcore_barrier()

        pltpu.sync_copy(
            acc.at[pl.ds(a0, ACC_T)],
            out_hbm.at[b, cb, pl.ds(a0, ACC_T)],
        )
        return carry

    lax.fori_loop(0, UNITS_PER_SC, _unit, 0)


@jax.jit
def kernel(updates, mask, output):
    del output
    upd_blk = (
        updates.reshape(B, HWIN, NCB, CB)
        .transpose(0, 2, 1, 3)
        .reshape(B, NCB, HWIN * CB)
    )
    mask_blk = (
        mask.astype(jnp.int32)
        .reshape(B, HWIN, NCB, CB)
        .transpose(0, 2, 1, 3)
        .reshape(B, NCB, HWIN * CB)
    )

    mesh = plsc.VectorSubcoreMesh(
        core_axis_name="c", subcore_axis_name="s",
        num_cores=NSC, num_subcores=NTILE,
    )
    run = pl.kernel(
        _sc_body,
        out_type=jax.ShapeDtypeStruct((B, NCB, ACC_N), jnp.float32),
        mesh=mesh,
        scratch_types=[
            pltpu.VMEM((NELEM_T,), jnp.float32),
            pltpu.VMEM((NELEM_T,), jnp.int32),
            pltpu.VMEM((NELEM_T,), jnp.int32),
            pltpu.VMEM((NELEM_T,), jnp.float32),
            pltpu.VMEM_SHARED((ACC_N,), jnp.float32),
        ],
    )
    out_blk = run(upd_blk, mask_blk)
    out = (
        out_blk.reshape(B, NCB, POUT, CB)
        .transpose(0, 2, 1, 3)
        .reshape(B, HOUT, WOUT, C)
    )
    return out


# speedup vs baseline: 15.8580x; 1.6298x over previous
import functools as _ft

import jax
import jax.numpy as jnp
from jax import lax
from jax.experimental import pallas as pl
from jax.experimental.pallas import tpu as pltpu
from jax.experimental.pallas import tpu_sc as plsc

B = 4
H = 192
W = 192
C = 96
HOUT = 384
WOUT = 384
HWIN = H * W
POUT = HOUT * WOUT
CB = 16
NCB = C // CB
NSC = 2
NTILE = 16
BPC = B // NSC
UNITS_PER_SC = BPC * NCB
NPOS_T = HWIN // NTILE
SPLIT = 2
HROWS = POUT // SPLIT
ACC_N = HROWS * CB
ACC_T = ACC_N // NTILE
LANES = 16
LOWMASK = (1 << 20) - 1
NCBP = 8

P1CH = 32
P1N = NPOS_T // P1CH
P1F = P1CH * CB
CELEM = 4608
NSUB = NPOS_T * CB // CELEM
PBR = 576
NPB = POUT // PBR
PBW = PBR * CB
ORT = POUT // NTILE
PBT = ORT // PBR
P3C = 64
P3S = PBR // P3C
P3W = P3C * CB


def _sc_body(upd_hbm, mask_hbm, out_hbm, blku, blkm, blko,
             out2d, fb2, mrows, fbi2, idxs, vals, m1d, u1d, acc):
    cid = lax.axis_index("c")
    sid = lax.axis_index("s")
    lanes = lax.iota(jnp.int32, LANES)

    def _p1(bloc, t, carry):
        b = cid * BPC + bloc
        p0 = sid * NPOS_T + t * P1CH
        pltpu.sync_copy(upd_hbm.at[b, pl.ds(p0, P1CH), :],
                        out2d.at[pl.ds(0, P1CH), :])
        pltpu.sync_copy(mask_hbm.at[b, pl.ds(p0, P1CH), :], mrows)

        def _as(i, c2):
            for cb in range(NCB):
                fb2[cb, pl.ds(i * LANES, LANES)] = out2d[i, pl.ds(cb * CB, CB)]
                fbi2[cb, pl.ds(i * LANES, LANES)] = mrows[i, pl.ds(cb * CB, CB)]
            return c2

        lax.fori_loop(0, P1CH, _as, 0)
        o0 = (p0 - sid * NPOS_T) * CB
        pltpu.sync_copy(fb2.at[:, pl.ds(0, P1F)],
                        blku.at[b, sid, :, pl.ds(o0, P1F)])
        pltpu.sync_copy(fbi2, blkm.at[b, sid, :, pl.ds(o0, P1F)])
        return carry

    for _bloc in range(BPC):
        lax.fori_loop(0, P1N, _ft.partial(_p1, _bloc), 0)
    plsc.subcore_barrier()

    a0 = sid * ACC_T

    def _unit(bloc, cb, carry):
        b = cid * BPC + bloc

        for q in range(SPLIT):
            qbase = q * ACC_N

            def _zv(i, c2):
                vals[pl.ds(i * LANES, LANES)] = jnp.zeros((LANES,), jnp.float32)
                return c2

            lax.fori_loop(0, CELEM // LANES, _zv, 0)
            for k in range(ACC_T // CELEM):
                pltpu.sync_copy(vals, acc.at[pl.ds(a0 + k * CELEM, CELEM)])
            plsc.subcore_barrier()

            for s in range(NSUB):
                e0 = s * CELEM
                pltpu.sync_copy(blkm.at[b, sid, cb, pl.ds(e0, CELEM)], m1d)
                pltpu.sync_copy(blku.at[b, sid, cb, pl.ds(e0, CELEM)], u1d)

                def _mk(i, c2):
                    m = m1d[pl.ds(i * LANES, LANES)]
                    v = u1d[pl.ds(i * LANES, LANES)]
                    n = lax.shift_right_logical(m, 5)
                    p = (
                        n.astype(jnp.float32) * jnp.float32(1.0 / 3.0)
                        + jnp.float32(0.5)
                    ).astype(jnp.int32)
                    rr = n - p * 3
                    p = p + lax.shift_right_arithmetic(rr, 31)
                    rel = lax.shift_left(p, 4) + lanes - qbase
                    inr = plsc.bitcast(rel, jnp.uint32) < jnp.uint32(ACC_N)
                    idxs[pl.ds(i * LANES, LANES)] = jnp.where(
                        inr, rel, rel & LOWMASK
                    )
                    vals[pl.ds(i * LANES, LANES)] = jnp.where(
                        inr, v, jnp.float32(0.0)
                    )
                    return c2

                lax.fori_loop(0, CELEM // LANES, _mk, 0)
                pltpu.sync_copy(vals, acc.at[idxs], add=True)
            plsc.subcore_barrier()

            pb0 = q * (HROWS // PBR) + sid * (ACC_T // CB // PBR)
            for j in range(ACC_T // PBW):
                pltpu.sync_copy(
                    acc.at[pl.ds(a0 + j * PBW, PBW)],
                    blko.at[b, pb0 + j, cb],
                )
        return carry

    for _bloc in range(BPC):
        lax.fori_loop(0, NCB, _ft.partial(_unit, _bloc), 0)
    plsc.subcore_barrier()

    def _p3(bloc, tp, ts, carry):
        b = cid * BPC + bloc
        r0 = sid * ORT + tp * PBR + ts * P3C
        pltpu.sync_copy(
            blko.at[b, sid * PBT + tp, :, pl.ds(ts * P3W, P3W)], fb2
        )

        def _as(i, c2):
            for cb in range(NCB):
                out2d[i, pl.ds(cb * CB, CB)] = fb2[cb, pl.ds(i * LANES, LANES)]
            return c2

        lax.fori_loop(0, P3C, _as, 0)
        pltpu.sync_copy(out2d, out_hbm.at[b, pl.ds(r0, P3C), :])
        return carry

    for _bloc in range(BPC):
        for _tp in range(PBT):
            lax.fori_loop(0, P3S, _ft.partial(_p3, _bloc, _tp), 0)


@jax.jit
def kernel(updates, mask, output):
    del output
    upd3 = updates.reshape(B, HWIN, C)
    mask3 = mask.astype(jnp.int32).reshape(B, HWIN, C)

    mesh = plsc.VectorSubcoreMesh(
        core_axis_name="c", subcore_axis_name="s",
        num_cores=NSC, num_subcores=NTILE,
    )
    run = pl.kernel(
        _sc_body,
        out_type=(
            jax.ShapeDtypeStruct((B, POUT, C), jnp.float32),
            jax.ShapeDtypeStruct((B, NTILE, NCBP, NPOS_T * CB), jnp.float32),
            jax.ShapeDtypeStruct((B, NTILE, NCBP, NPOS_T * CB), jnp.int32),
            jax.ShapeDtypeStruct((B, NPB, NCBP, PBW), jnp.float32),
        ),
        mesh=mesh,
        scratch_types=[
            pltpu.VMEM((P3C, C), jnp.float32),
            pltpu.VMEM((NCBP, P3W), jnp.float32),
            pltpu.VMEM((P1CH, C), jnp.int32),
            pltpu.VMEM((NCBP, P1F), jnp.int32),
            pltpu.VMEM((CELEM,), jnp.int32),
            pltpu.VMEM((CELEM,), jnp.float32),
            pltpu.VMEM((CELEM,), jnp.int32),
            pltpu.VMEM((CELEM,), jnp.float32),
            pltpu.VMEM_SHARED((ACC_N,), jnp.float32),
        ],
    )
    out, _, _, _ = run(upd3, mask3)
    return out.reshape(B, HOUT, WOUT, C)
